# baseline (speedup 1.0000x reference)
import jax
import jax.numpy as jnp
from jax import lax
from jax.experimental import pallas as pl
from jax.experimental.pallas import tpu as pltpu
from jax.experimental.pallas import tpu_sc as plsc

BATCH = 16384
EMB = 128
NC = 2
NS = 16
NW = NC * NS
ROWS_PER_W = BATCH // NW
CHUNK = 128
N_CHUNKS = ROWS_PER_W // CHUNK
LANES = 16
SEGS = EMB // LANES


def _dot_kernel(u_hbm, i_hbm, ut_hbm, it_hbm, out_hbm,
                uidx_v, iidx_v,
                urows0, urows1, irows0, irows1,
                outv,
                sem_u0, sem_u1, sem_i0, sem_i1, sem_x0, sem_x1):
    wid = lax.axis_index("s") * NC + lax.axis_index("c")
    base = wid * ROWS_PER_W

    cu = pltpu.async_copy(u_hbm.at[pl.ds(wid * N_CHUNKS, N_CHUNKS)], uidx_v, sem_x0)
    ci = pltpu.async_copy(i_hbm.at[pl.ds(wid * N_CHUNKS, N_CHUNKS)], iidx_v, sem_x1)

    zeros = jnp.zeros((LANES,), jnp.float32)

    @plsc.parallel_loop(0, ROWS_PER_W // LANES, step=1)
    def _z(g):
        outv[pl.ds(g * LANES, LANES)] = zeros

    cu.wait()
    ci.wait()

    pltpu.async_copy(ut_hbm.at[uidx_v.at[0]], urows0, sem_u0)
    pltpu.async_copy(it_hbm.at[iidx_v.at[0]], irows0, sem_i0)
    pltpu.async_copy(ut_hbm.at[uidx_v.at[1]], urows1, sem_u1)
    pltpu.async_copy(it_hbm.at[iidx_v.at[1]], irows1, sem_i1)

    def compute_chunk(c, ur, ir):
        cbase = c * CHUNK

        @plsc.parallel_loop(0, CHUNK, step=1)
        def _row(r):
            m = [ur[r, pl.ds(s * LANES, LANES)] * ir[r, pl.ds(s * LANES, LANES)]
                 for s in range(SEGS)]
            while len(m) > 1:
                m = [m[2 * k] + m[2 * k + 1] for k in range(len(m) // 2)]
            plsc.addupdate_scatter(
                outv, [jnp.full((LANES,), cbase + r, jnp.int32)], m[0])

    def pair_body(p, _):
        c0 = 2 * p
        pltpu.make_async_copy(ut_hbm.at[pl.ds(0, CHUNK)], urows0, sem_u0).wait()
        pltpu.make_async_copy(it_hbm.at[pl.ds(0, CHUNK)], irows0, sem_i0).wait()
        compute_chunk(c0, urows0, irows0)

        @pl.when(c0 + 2 < N_CHUNKS)
        def _():
            pltpu.async_copy(ut_hbm.at[uidx_v.at[c0 + 2]], urows0, sem_u0)
            pltpu.async_copy(it_hbm.at[iidx_v.at[c0 + 2]], irows0, sem_i0)

        pltpu.make_async_copy(ut_hbm.at[pl.ds(0, CHUNK)], urows1, sem_u1).wait()
        pltpu.make_async_copy(it_hbm.at[pl.ds(0, CHUNK)], irows1, sem_i1).wait()
        compute_chunk(c0 + 1, urows1, irows1)

        @pl.when(c0 + 3 < N_CHUNKS)
        def _():
            pltpu.async_copy(ut_hbm.at[uidx_v.at[c0 + 3]], urows1, sem_u1)
            pltpu.async_copy(it_hbm.at[iidx_v.at[c0 + 3]], irows1, sem_i1)

        return 0

    lax.fori_loop(0, N_CHUNKS // 2, pair_body, 0)
    pltpu.sync_copy(outv, out_hbm.at[pl.ds(base, ROWS_PER_W)])


@jax.jit
def _run(u, i, user_table, item_table):
    mesh = plsc.VectorSubcoreMesh(core_axis_name="c", subcore_axis_name="s")
    f = pl.kernel(
        _dot_kernel,
        out_type=jax.ShapeDtypeStruct((BATCH,), jnp.float32),
        mesh=mesh,
        compiler_params=pltpu.CompilerParams(needs_layout_passes=False,
                                             disable_bounds_checks=True,
                                             disable_semaphore_checks=True),
        scratch_types=[
            pltpu.VMEM((N_CHUNKS, CHUNK), jnp.int32),
            pltpu.VMEM((N_CHUNKS, CHUNK), jnp.int32),
            pltpu.VMEM((CHUNK, EMB), jnp.float32),
            pltpu.VMEM((CHUNK, EMB), jnp.float32),
            pltpu.VMEM((CHUNK, EMB), jnp.float32),
            pltpu.VMEM((CHUNK, EMB), jnp.float32),
            pltpu.VMEM((ROWS_PER_W,), jnp.float32),
            pltpu.SemaphoreType.DMA,
            pltpu.SemaphoreType.DMA,
            pltpu.SemaphoreType.DMA,
            pltpu.SemaphoreType.DMA,
            pltpu.SemaphoreType.DMA,
            pltpu.SemaphoreType.DMA,
        ],
    )
    return f(u.reshape(BATCH // CHUNK, CHUNK), i.reshape(BATCH // CHUNK, CHUNK),
             user_table, item_table)


def kernel(u, i, user_table, item_table):
    return _run(u.astype(jnp.int32), i.astype(jnp.int32), user_table, item_table)


# speedup vs baseline: 1.1495x; 1.1495x over previous
import jax
import jax.numpy as jnp
from jax import lax
from jax.experimental import pallas as pl
from jax.experimental.pallas import tpu as pltpu
from jax.experimental.pallas import tpu_sc as plsc

BATCH = 16384
EMB = 128
NC = 2
NS = 16
NW = NC * NS
ROWS_PER_W = BATCH // NW
CHUNK = 128
N_CHUNKS = ROWS_PER_W // CHUNK
LANES = 16
SEGS = EMB // LANES
PSTRIDE = LANES + 1


def _dot_kernel(u_hbm, i_hbm, ut_hbm, it_hbm, out_hbm,
                uidx_v, iidx_v,
                urows0, urows1, irows0, irows1,
                part_v, outv,
                sem_u0, sem_u1, sem_i0, sem_i1, sem_x0, sem_x1):
    wid = lax.axis_index("s") * NC + lax.axis_index("c")
    base = wid * ROWS_PER_W

    lane = lax.iota(jnp.int32, LANES)
    cols = [jnp.full((LANES,), j, jnp.int32) for j in range(LANES)]

    cu = pltpu.async_copy(u_hbm.at[pl.ds(wid * N_CHUNKS, N_CHUNKS)], uidx_v, sem_x0)
    ci = pltpu.async_copy(i_hbm.at[pl.ds(wid * N_CHUNKS, N_CHUNKS)], iidx_v, sem_x1)
    cu.wait()
    ci.wait()

    pltpu.async_copy(ut_hbm.at[uidx_v.at[0]], urows0, sem_u0)
    pltpu.async_copy(it_hbm.at[iidx_v.at[0]], irows0, sem_i0)
    pltpu.async_copy(ut_hbm.at[uidx_v.at[1]], urows1, sem_u1)
    pltpu.async_copy(it_hbm.at[iidx_v.at[1]], irows1, sem_i1)

    def compute_chunk(c, ur, ir):
        @plsc.parallel_loop(0, CHUNK, step=1)
        def _row(r):
            m = [ur[r, pl.ds(s * LANES, LANES)] * ir[r, pl.ds(s * LANES, LANES)]
                 for s in range(SEGS)]
            while len(m) > 1:
                m = [m[2 * k] + m[2 * k + 1] for k in range(len(m) // 2)]
            part_v[r, pl.ds(0, LANES)] = m[0]

        cbase = c * CHUNK

        @plsc.parallel_loop(0, CHUNK // LANES, step=1)
        def _grp(g):
            rows = g * LANES + lane
            t = [plsc.load_gather(part_v, [rows, cols[j]]) for j in range(LANES)]
            while len(t) > 1:
                t = [t[2 * k] + t[2 * k + 1] for k in range(len(t) // 2)]
            outv[pl.ds(cbase + g * LANES, LANES)] = t[0]

    def pair_body(p, _):
        c0 = 2 * p
        pltpu.make_async_copy(ut_hbm.at[pl.ds(0, CHUNK)], urows0, sem_u0).wait()
        pltpu.make_async_copy(it_hbm.at[pl.ds(0, CHUNK)], irows0, sem_i0).wait()
        compute_chunk(c0, urows0, irows0)

        @pl.when(c0 + 2 < N_CHUNKS)
        def _():
            pltpu.async_copy(ut_hbm.at[uidx_v.at[c0 + 2]], urows0, sem_u0)
            pltpu.async_copy(it_hbm.at[iidx_v.at[c0 + 2]], irows0, sem_i0)

        pltpu.make_async_copy(ut_hbm.at[pl.ds(0, CHUNK)], urows1, sem_u1).wait()
        pltpu.make_async_copy(it_hbm.at[pl.ds(0, CHUNK)], irows1, sem_i1).wait()
        compute_chunk(c0 + 1, urows1, irows1)

        @pl.when(c0 + 3 < N_CHUNKS)
        def _():
            pltpu.async_copy(ut_hbm.at[uidx_v.at[c0 + 3]], urows1, sem_u1)
            pltpu.async_copy(it_hbm.at[iidx_v.at[c0 + 3]], irows1, sem_i1)

        return 0

    lax.fori_loop(0, N_CHUNKS // 2, pair_body, 0)
    pltpu.sync_copy(outv, out_hbm.at[pl.ds(base, ROWS_PER_W)])


@jax.jit
def _run(u, i, user_table, item_table):
    mesh = plsc.VectorSubcoreMesh(core_axis_name="c", subcore_axis_name="s")
    f = pl.kernel(
        _dot_kernel,
        out_type=jax.ShapeDtypeStruct((BATCH,), jnp.float32),
        mesh=mesh,
        compiler_params=pltpu.CompilerParams(needs_layout_passes=False,
                                             disable_bounds_checks=True,
                                             disable_semaphore_checks=True),
        scratch_types=[
            pltpu.VMEM((N_CHUNKS, CHUNK), jnp.int32),
            pltpu.VMEM((N_CHUNKS, CHUNK), jnp.int32),
            pltpu.VMEM((CHUNK, EMB), jnp.float32),
            pltpu.VMEM((CHUNK, EMB), jnp.float32),
            pltpu.VMEM((CHUNK, EMB), jnp.float32),
            pltpu.VMEM((CHUNK, EMB), jnp.float32),
            pltpu.VMEM((CHUNK, PSTRIDE), jnp.float32),
            pltpu.VMEM((ROWS_PER_W,), jnp.float32),
            pltpu.SemaphoreType.DMA,
            pltpu.SemaphoreType.DMA,
            pltpu.SemaphoreType.DMA,
            pltpu.SemaphoreType.DMA,
            pltpu.SemaphoreType.DMA,
            pltpu.SemaphoreType.DMA,
        ],
    )
    return f(u.reshape(BATCH // CHUNK, CHUNK), i.reshape(BATCH // CHUNK, CHUNK),
             user_table, item_table)


def kernel(u, i, user_table, item_table):
    return _run(u.astype(jnp.int32), i.astype(jnp.int32), user_table, item_table)


# speedup vs baseline: 1.1554x; 1.0051x over previous
import jax
import jax.numpy as jnp
from jax import lax
from jax.experimental import pallas as pl
from jax.experimental.pallas import tpu as pltpu
from jax.experimental.pallas import tpu_sc as plsc

BATCH = 16384
EMB = 128
NC = 2
NS = 16
NW = NC * NS
ROWS_PER_W = BATCH // NW
CHUNK = 64
N_CHUNKS = ROWS_PER_W // CHUNK
NBUF = 4
LANES = 16
SEGS = EMB // LANES
PSTRIDE = LANES + 1


def _dot_kernel(u_hbm, i_hbm, ut_hbm, it_hbm, out_hbm,
                uidx_v, iidx_v,
                ur0, ur1, ur2, ur3, ir0, ir1, ir2, ir3,
                part_v, outv,
                su0, su1, su2, su3, si0, si1, si2, si3, sx0, sx1):
    wid = lax.axis_index("s") * NC + lax.axis_index("c")
    base = wid * ROWS_PER_W

    urows = [ur0, ur1, ur2, ur3]
    irows = [ir0, ir1, ir2, ir3]
    sem_u = [su0, su1, su2, su3]
    sem_i = [si0, si1, si2, si3]

    lane = lax.iota(jnp.int32, LANES)
    cols = [jnp.full((LANES,), j, jnp.int32) for j in range(LANES)]

    cu = pltpu.async_copy(u_hbm.at[pl.ds(wid * N_CHUNKS, N_CHUNKS)], uidx_v, sx0)
    ci = pltpu.async_copy(i_hbm.at[pl.ds(wid * N_CHUNKS, N_CHUNKS)], iidx_v, sx1)
    cu.wait()
    ci.wait()

    for b in range(NBUF):
        pltpu.async_copy(ut_hbm.at[uidx_v.at[b]], urows[b], sem_u[b])
        pltpu.async_copy(it_hbm.at[iidx_v.at[b]], irows[b], sem_i[b])

    def compute_chunk(c, ur, ir):
        @plsc.parallel_loop(0, CHUNK, step=1)
        def _row(r):
            m = [ur[r, pl.ds(s * LANES, LANES)] * ir[r, pl.ds(s * LANES, LANES)]
                 for s in range(SEGS)]
            while len(m) > 1:
                m = [m[2 * k] + m[2 * k + 1] for k in range(len(m) // 2)]
            part_v[r, pl.ds(0, LANES)] = m[0]

        cbase = c * CHUNK

        @plsc.parallel_loop(0, CHUNK // LANES, step=1)
        def _grp(g):
            rows = g * LANES + lane
            t = [plsc.load_gather(part_v, [rows, cols[j]]) for j in range(LANES)]
            while len(t) > 1:
                t = [t[2 * k] + t[2 * k + 1] for k in range(len(t) // 2)]
            outv[pl.ds(cbase + g * LANES, LANES)] = t[0]

    def quad_body(p, _):
        c0 = NBUF * p
        for b in range(NBUF):
            c = c0 + b
            pltpu.make_async_copy(ut_hbm.at[pl.ds(0, CHUNK)], urows[b], sem_u[b]).wait()
            pltpu.make_async_copy(it_hbm.at[pl.ds(0, CHUNK)], irows[b], sem_i[b]).wait()
            compute_chunk(c, urows[b], irows[b])

            @pl.when(c + NBUF < N_CHUNKS)
            def _():
                pltpu.async_copy(ut_hbm.at[uidx_v.at[c + NBUF]], urows[b], sem_u[b])
                pltpu.async_copy(it_hbm.at[iidx_v.at[c + NBUF]], irows[b], sem_i[b])

        return 0

    lax.fori_loop(0, N_CHUNKS // NBUF, quad_body, 0)
    pltpu.sync_copy(outv, out_hbm.at[pl.ds(base, ROWS_PER_W)])


@jax.jit
def _run(u, i, user_table, item_table):
    mesh = plsc.VectorSubcoreMesh(core_axis_name="c", subcore_axis_name="s")
    f = pl.kernel(
        _dot_kernel,
        out_type=jax.ShapeDtypeStruct((BATCH,), jnp.float32),
        mesh=mesh,
        compiler_params=pltpu.CompilerParams(needs_layout_passes=False,
                                             disable_bounds_checks=True,
                                             disable_semaphore_checks=True),
        scratch_types=(
            [pltpu.VMEM((N_CHUNKS, CHUNK), jnp.int32)] * 2
            + [pltpu.VMEM((CHUNK, EMB), jnp.float32)] * (2 * NBUF)
            + [pltpu.VMEM((CHUNK, PSTRIDE), jnp.float32),
               pltpu.VMEM((ROWS_PER_W,), jnp.float32)]
            + [pltpu.SemaphoreType.DMA] * (2 * NBUF + 2)
        ),
    )
    return f(u.reshape(BATCH // CHUNK, CHUNK), i.reshape(BATCH // CHUNK, CHUNK),
             user_table, item_table)


def kernel(u, i, user_table, item_table):
    return _run(u.astype(jnp.int32), i.astype(jnp.int32), user_table, item_table)


# speedup vs baseline: 1.1572x; 1.0016x over previous
import jax
import jax.numpy as jnp
from jax import lax
from jax.experimental import pallas as pl
from jax.experimental.pallas import tpu as pltpu
from jax.experimental.pallas import tpu_sc as plsc

BATCH = 16384
EMB = 128
NC = 2
NS = 16
NW = NC * NS
ROWS_PER_W = BATCH // NW
CHUNK = 32
N_CHUNKS = ROWS_PER_W // CHUNK
NBUF = 8
LANES = 16
SEGS = EMB // LANES
PSTRIDE = LANES + 1


def _dot_kernel(u_hbm, i_hbm, ut_hbm, it_hbm, out_hbm,
                uidx_v, iidx_v,
                ur0, ur1, ur2, ur3, ur4, ur5, ur6, ur7,
                ir0, ir1, ir2, ir3, ir4, ir5, ir6, ir7,
                part_v, outv,
                su0, su1, su2, su3, su4, su5, su6, su7,
                si0, si1, si2, si3, si4, si5, si6, si7, sx0, sx1):
    wid = lax.axis_index("s") * NC + lax.axis_index("c")
    base = wid * ROWS_PER_W

    urows = [ur0, ur1, ur2, ur3, ur4, ur5, ur6, ur7]
    irows = [ir0, ir1, ir2, ir3, ir4, ir5, ir6, ir7]
    sem_u = [su0, su1, su2, su3, su4, su5, su6, su7]
    sem_i = [si0, si1, si2, si3, si4, si5, si6, si7]

    lane = lax.iota(jnp.int32, LANES)
    cols = [jnp.full((LANES,), j, jnp.int32) for j in range(LANES)]

    cu = pltpu.async_copy(u_hbm.at[pl.ds(wid * N_CHUNKS, N_CHUNKS)], uidx_v, sx0)
    ci = pltpu.async_copy(i_hbm.at[pl.ds(wid * N_CHUNKS, N_CHUNKS)], iidx_v, sx1)
    cu.wait()
    ci.wait()

    for b in range(NBUF):
        pltpu.async_copy(ut_hbm.at[uidx_v.at[b]], urows[b], sem_u[b])
        pltpu.async_copy(it_hbm.at[iidx_v.at[b]], irows[b], sem_i[b])

    def compute_chunk(c, ur, ir):
        @plsc.parallel_loop(0, CHUNK, step=1)
        def _row(r):
            m = [ur[r, pl.ds(s * LANES, LANES)] * ir[r, pl.ds(s * LANES, LANES)]
                 for s in range(SEGS)]
            while len(m) > 1:
                m = [m[2 * k] + m[2 * k + 1] for k in range(len(m) // 2)]
            part_v[r, pl.ds(0, LANES)] = m[0]

        cbase = c * CHUNK

        @plsc.parallel_loop(0, CHUNK // LANES, step=1)
        def _grp(g):
            rows = g * LANES + lane
            t = [plsc.load_gather(part_v, [rows, cols[j]]) for j in range(LANES)]
            while len(t) > 1:
                t = [t[2 * k] + t[2 * k + 1] for k in range(len(t) // 2)]
            outv[pl.ds(cbase + g * LANES, LANES)] = t[0]

    def quad_body(p, _):
        c0 = NBUF * p
        for b in range(NBUF):
            c = c0 + b
            pltpu.make_async_copy(ut_hbm.at[pl.ds(0, CHUNK)], urows[b], sem_u[b]).wait()
            pltpu.make_async_copy(it_hbm.at[pl.ds(0, CHUNK)], irows[b], sem_i[b]).wait()
            compute_chunk(c, urows[b], irows[b])

            @pl.when(c + NBUF < N_CHUNKS)
            def _():
                pltpu.async_copy(ut_hbm.at[uidx_v.at[c + NBUF]], urows[b], sem_u[b])
                pltpu.async_copy(it_hbm.at[iidx_v.at[c + NBUF]], irows[b], sem_i[b])

        return 0

    lax.fori_loop(0, N_CHUNKS // NBUF, quad_body, 0)
    pltpu.sync_copy(outv, out_hbm.at[pl.ds(base, ROWS_PER_W)])


@jax.jit
def _run(u, i, user_table, item_table):
    mesh = plsc.VectorSubcoreMesh(core_axis_name="c", subcore_axis_name="s")
    f = pl.kernel(
        _dot_kernel,
        out_type=jax.ShapeDtypeStruct((BATCH,), jnp.float32),
        mesh=mesh,
        compiler_params=pltpu.CompilerParams(needs_layout_passes=False,
                                             disable_bounds_checks=True,
                                             disable_semaphore_checks=True),
        scratch_types=(
            [pltpu.VMEM((N_CHUNKS, CHUNK), jnp.int32)] * 2
            + [pltpu.VMEM((CHUNK, EMB), jnp.float32)] * (2 * NBUF)
            + [pltpu.VMEM((CHUNK, PSTRIDE), jnp.float32),
               pltpu.VMEM((ROWS_PER_W,), jnp.float32)]
            + [pltpu.SemaphoreType.DMA] * (2 * NBUF + 2)
        ),
    )
    return f(u.reshape(BATCH // CHUNK, CHUNK), i.reshape(BATCH // CHUNK, CHUNK),
             user_table, item_table)


def kernel(u, i, user_table, item_table):
    return _run(u.astype(jnp.int32), i.astype(jnp.int32), user_table, item_table)
